# baseline (speedup 1.0000x reference)
import functools
import math

import jax
import jax.numpy as jnp
from jax import lax
from jax.experimental import pallas as pl
from jax.experimental.pallas import tpu as pltpu

_H = 768


def _gelu(x):
    return 0.5 * x * (1.0 + lax.erf(x * (1.0 / math.sqrt(2.0))))


def _fused_block(node_ref, masif_ref, gbs_ref,
                 mw1_ref, mb1_ref, mwf_ref,
                 gw1_ref, gb1_ref, gwf_ref,
                 tokb_ref, nw_ref, nb_ref, o_ref, *, n, f):
    gb = node_ref.shape[0]
    h = _H

    mh = _gelu(jnp.dot(masif_ref[...].astype(jnp.bfloat16), mw1_ref[...],
                       preferred_element_type=jnp.float32) + mb1_ref[...])
    gh = _gelu(jnp.dot(gbs_ref[...].astype(jnp.bfloat16), gw1_ref[...],
                       preferred_element_type=jnp.float32) + gb1_ref[...])
    tok = (jnp.dot(mh.astype(jnp.bfloat16), mwf_ref[...],
                   preferred_element_type=jnp.float32)
           + jnp.dot(gh.astype(jnp.bfloat16), gwf_ref[...],
                     preferred_element_type=jnp.float32)
           + tokb_ref[...])

    nflat = node_ref[...].astype(jnp.bfloat16)
    nstack = jnp.stack([nflat[:, i * f:(i + 1) * f] for i in range(n)],
                       axis=1)
    nodes = jnp.dot(nstack.reshape(gb * n, f),
                    nw_ref[...], preferred_element_type=jnp.float32)
    nodes = nodes.reshape(gb, n, h) + nb_ref[...].reshape(1, 1, h)

    o_ref[...] = jnp.concatenate([tok.reshape(gb, 1, h), nodes], axis=1)


def _resident(shape):
    nd = len(shape)
    return pl.BlockSpec(shape, lambda i: (0,) * nd)


def kernel(node_w, node_b, graph_token,
           masif_w1, masif_b1, masif_w2, masif_b2,
           gb_w1, gb_b1, gb_w2, gb_b2,
           fuse_w, fuse_b,
           node_feat, masif_desc_straight, gbscore):
    h = _H
    g, n, f = node_feat.shape
    m = masif_desc_straight.shape[1]
    dg = gbscore.shape[1]

    gb = min(g, 64)
    while g % gb:
        gb -= 1

    fwm = fuse_w[h:2 * h]
    fwg = fuse_w[2 * h:3 * h]
    tok_bias = (graph_token @ fuse_w[:h] + fuse_b
                + masif_b2 @ fwm + gb_b2 @ fwg)
    mwf = (masif_w2 @ fwm).astype(jnp.bfloat16)
    gwf = (gb_w2 @ fwg).astype(jnp.bfloat16)
    mw1s = jnp.tile(masif_w1 * (1.0 / m), (m, 1)).astype(jnp.bfloat16)
    gw1b = gb_w1.astype(jnp.bfloat16)
    nwb = node_w.astype(jnp.bfloat16)

    dmf = m * masif_desc_straight.shape[2]
    masif_flat = masif_desc_straight.reshape(g, dmf)
    node_flat = node_feat.reshape(g, n * f)

    w_args = (mw1s, masif_b1, mwf, gw1b, gb_b1, gwf, tok_bias, nwb, node_b)
    w_specs = [_resident(tuple(w.shape)) for w in w_args]

    return pl.pallas_call(
        functools.partial(_fused_block, n=n, f=f),
        out_shape=jax.ShapeDtypeStruct((g, n + 1, h), jnp.float32),
        grid=(g // gb,),
        in_specs=[
            pl.BlockSpec((gb, n * f), lambda i: (i, 0)),
            pl.BlockSpec((gb, dmf), lambda i: (i, 0)),
            pl.BlockSpec((gb, dg), lambda i: (i, 0)),
        ] + w_specs,
        out_specs=pl.BlockSpec((gb, n + 1, h), lambda i: (i, 0, 0)),
        compiler_params=pltpu.CompilerParams(
            dimension_semantics=("parallel",),
            vmem_limit_bytes=(64 << 20) * 4 // 5),
    )(node_flat, masif_flat, gbscore, *w_args)


# speedup vs baseline: 3.4652x; 3.4652x over previous
import functools
import math

import jax
import jax.numpy as jnp
from jax import lax
from jax.experimental import pallas as pl
from jax.experimental.pallas import tpu as pltpu

_H = 768


def _gelu(x):
    return 0.5 * x * (1.0 + lax.erf(x * (1.0 / math.sqrt(2.0))))


def _dot_t(lhs_t, rhs):
    return lax.dot_general(lhs_t, rhs, (((0,), (0,)), ((), ())),
                           preferred_element_type=jnp.float32)


def _fused_block(node_ref, masif_ref, gbs_ref,
                 mw1_ref, mb1_ref, mwf_ref,
                 gw1_ref, gb1_ref, gwf_ref,
                 tokb_ref, nw_ref, nb_ref, o_ref, *, n):
    msum = jnp.sum(masif_ref[...], axis=0)
    mh = _gelu(_dot_t(msum.astype(jnp.bfloat16), mw1_ref[...]) + mb1_ref[...])
    gh = _gelu(_dot_t(gbs_ref[...].astype(jnp.bfloat16), gw1_ref[...])
               + gb1_ref[...])
    tok = (jnp.dot(mh.astype(jnp.bfloat16), mwf_ref[...],
                   preferred_element_type=jnp.float32)
           + jnp.dot(gh.astype(jnp.bfloat16), gwf_ref[...],
                     preferred_element_type=jnp.float32)
           + tokb_ref[...])
    o_ref[0, :, :] = tok

    nfeat = node_ref[...].astype(jnp.bfloat16)
    nw = nw_ref[...]
    nb = nb_ref[...]
    for i in range(n):
        o_ref[1 + i, :, :] = _dot_t(nfeat[:, i, :], nw) + nb


def kernel(node_w, node_b, graph_token,
           masif_w1, masif_b1, masif_w2, masif_b2,
           gb_w1, gb_b1, gb_w2, gb_b2,
           fuse_w, fuse_b,
           node_feat, masif_desc_straight, gbscore):
    h = _H
    g, n, f = node_feat.shape
    m, dm = masif_desc_straight.shape[1], masif_desc_straight.shape[2]
    dg = gbscore.shape[1]

    gb = min(g, 128)
    while g % gb:
        gb -= 1
    hb = h // 2

    fwm = fuse_w[h:2 * h]
    fwg = fuse_w[2 * h:3 * h]
    tok_bias = (graph_token @ fuse_w[:h] + fuse_b
                + masif_b2 @ fwm + gb_b2 @ fwg)
    mwf = (masif_w2 @ fwm).astype(jnp.bfloat16)
    gwf = (gb_w2 @ fwg).astype(jnp.bfloat16)
    mw1s = (masif_w1 * (1.0 / m)).astype(jnp.bfloat16)
    gw1b = gb_w1.astype(jnp.bfloat16)
    nwb = node_w.astype(jnp.bfloat16)

    node_t = node_feat.transpose(2, 1, 0)
    masif_t = masif_desc_straight.transpose(1, 2, 0)
    gbs_t = gbscore.transpose(1, 0)

    def res(shape):
        nd = len(shape)
        return pl.BlockSpec(shape, lambda i, j: (0,) * nd)

    def hblk(shape):
        bs = tuple(shape[:-1]) + (hb,)
        nd = len(shape)
        return pl.BlockSpec(bs, lambda i, j: (0,) * (nd - 1) + (j,))

    w_args = (mw1s, masif_b1, mwf, gw1b, gb_b1, gwf, tok_bias, nwb, node_b)
    w_specs = [res(mw1s.shape), res(masif_b1.shape), hblk(mwf.shape),
               res(gw1b.shape), res(gb_b1.shape), hblk(gwf.shape),
               hblk(tok_bias.shape), hblk(nwb.shape), hblk(node_b.shape)]

    out_t = pl.pallas_call(
        functools.partial(_fused_block, n=n),
        out_shape=jax.ShapeDtypeStruct((n + 1, g, h), jnp.float32),
        grid=(g // gb, h // hb),
        in_specs=[
            pl.BlockSpec((f, n, gb), lambda i, j: (0, 0, i)),
            pl.BlockSpec((m, dm, gb), lambda i, j: (0, 0, i)),
            pl.BlockSpec((dg, gb), lambda i, j: (0, i)),
        ] + w_specs,
        out_specs=pl.BlockSpec((n + 1, gb, hb), lambda i, j: (0, i, j)),
        compiler_params=pltpu.CompilerParams(
            dimension_semantics=("parallel", "arbitrary"),
            vmem_limit_bytes=(64 << 20) * 4 // 5),
    )(node_t, masif_t, gbs_t, *w_args)

    return out_t.transpose(1, 0, 2)


# speedup vs baseline: 3.5316x; 1.0192x over previous
import functools
import math

import jax
import jax.numpy as jnp
from jax import lax
from jax.experimental import pallas as pl
from jax.experimental.pallas import tpu as pltpu

_H = 768


def _gelu(x):
    return 0.5 * x * (1.0 + lax.erf(x * (1.0 / math.sqrt(2.0))))


def _dot_t(lhs_t, rhs):
    return lax.dot_general(lhs_t, rhs, (((0,), (0,)), ((), ())),
                           preferred_element_type=jnp.float32)


def _fused_block(node_ref, masif_ref, gbs_ref,
                 mw1_ref, mb1_ref, mwf_ref,
                 gw1_ref, gb1_ref, gwf_ref,
                 tokb_ref, nw_ref, nb_ref, o_ref, *, n):
    msum = jnp.sum(masif_ref[...], axis=0)
    mh = _gelu(_dot_t(msum.astype(jnp.bfloat16), mw1_ref[...]) + mb1_ref[...])
    gh = _gelu(_dot_t(gbs_ref[...].astype(jnp.bfloat16), gw1_ref[...])
               + gb1_ref[...])
    tok = (jnp.dot(mh.astype(jnp.bfloat16), mwf_ref[...],
                   preferred_element_type=jnp.float32)
           + jnp.dot(gh.astype(jnp.bfloat16), gwf_ref[...],
                     preferred_element_type=jnp.float32)
           + tokb_ref[...])
    o_ref[0, :, :] = tok

    nfeat = node_ref[...].astype(jnp.bfloat16)
    nw = nw_ref[...]
    nb = nb_ref[...]
    for i in range(n):
        o_ref[1 + i, :, :] = _dot_t(nfeat[:, i, :], nw) + nb


def kernel(node_w, node_b, graph_token,
           masif_w1, masif_b1, masif_w2, masif_b2,
           gb_w1, gb_b1, gb_w2, gb_b2,
           fuse_w, fuse_b,
           node_feat, masif_desc_straight, gbscore):
    h = _H
    g, n, f = node_feat.shape
    m, dm = masif_desc_straight.shape[1], masif_desc_straight.shape[2]
    dg = gbscore.shape[1]

    gb = min(g, 128)
    while g % gb:
        gb -= 1

    fwm = fuse_w[h:2 * h]
    fwg = fuse_w[2 * h:3 * h]
    tok_bias = (graph_token @ fuse_w[:h] + fuse_b
                + masif_b2 @ fwm + gb_b2 @ fwg)
    mwf = (masif_w2 @ fwm).astype(jnp.bfloat16)
    gwf = (gb_w2 @ fwg).astype(jnp.bfloat16)
    mw1s = (masif_w1 * (1.0 / m)).astype(jnp.bfloat16)
    gw1b = gb_w1.astype(jnp.bfloat16)
    nwb = node_w.astype(jnp.bfloat16)

    node_t = node_feat.transpose(2, 1, 0)
    masif_t = masif_desc_straight.transpose(1, 2, 0)
    gbs_t = gbscore.transpose(1, 0)

    def res(shape):
        nd = len(shape)
        return pl.BlockSpec(shape, lambda i: (0,) * nd)

    w_args = (mw1s, masif_b1, mwf, gw1b, gb_b1, gwf, tok_bias, nwb, node_b)
    w_specs = [res(tuple(w.shape)) for w in w_args]

    out_t = pl.pallas_call(
        functools.partial(_fused_block, n=n),
        out_shape=jax.ShapeDtypeStruct((n + 1, g, h), jnp.float32),
        grid=(g // gb,),
        in_specs=[
            pl.BlockSpec((f, n, gb), lambda i: (0, 0, i)),
            pl.BlockSpec((m, dm, gb), lambda i: (0, 0, i)),
            pl.BlockSpec((dg, gb), lambda i: (0, i)),
        ] + w_specs,
        out_specs=pl.BlockSpec((n + 1, gb, h), lambda i: (0, i, 0)),
        compiler_params=pltpu.CompilerParams(
            dimension_semantics=("parallel",),
            vmem_limit_bytes=63 << 20),
    )(node_t, masif_t, gbs_t, *w_args)

    return out_t.transpose(1, 0, 2)
